# baseline (speedup 1.0000x reference)
import functools

import jax
import jax.numpy as jnp
from jax import lax
from jax.experimental import pallas as pl
from jax.experimental.pallas import tpu as pltpu
from jax.experimental.pallas import tpu_sc as plsc

N = 50000
F = 256
B = 1024
D = 4

NP = 50176
BLK = 512
NBLK = NP // BLK
BP = 2048
BACC = 1152
NT = 16
CHUNK = NP // NT
NV = CHUNK // 16
HALF = CHUNK // 2
KR = 56
GPT = HALF // KR
IDXR = 32
RPT = BACC // NT

_SENT = BP - 1


def _k1_body(nodes_ref, wv_ref, bv_ref, wal_ref, hv_ref, at_ref):
  x = nodes_ref[...]
  h = jnp.dot(x, wv_ref[...], preferred_element_type=jnp.float32)
  h = jnp.maximum(h + bv_ref[...], 0.0)
  hv_ref[...] = h
  at_ref[...] = lax.dot_general(
      wal_ref[...], h, (((1,), (1,)), ((), ())),
      preferred_element_type=jnp.float32)


def _k1(nodes_p, wv, bv2, wal8):
  return pl.pallas_call(
      _k1_body,
      grid=(NBLK,),
      in_specs=[
          pl.BlockSpec((BLK, F), lambda i: (i, 0)),
          pl.BlockSpec((F, F), lambda i: (0, 0)),
          pl.BlockSpec((1, F), lambda i: (0, 0)),
          pl.BlockSpec((8, F), lambda i: (0, 0)),
      ],
      out_specs=[
          pl.BlockSpec((BLK, F), lambda i: (i, 0)),
          pl.BlockSpec((8, BLK), lambda i: (0, i)),
      ],
      out_shape=[
          jax.ShapeDtypeStruct((NP, F), jnp.float32),
          jax.ShapeDtypeStruct((8, NP), jnp.float32),
      ],
      compiler_params=pltpu.CompilerParams(
          dimension_semantics=("arbitrary",)),
  )(nodes_p, wv, bv2, wal8)


_IOTA = None


def _shifted(buf_ref, iota, shift):
  return plsc.load_gather(buf_ref, [iota + (16 - shift)])


def _seg_combine(ids, vals, ibuf, vbuf, iota, is_max):
  neutral = jnp.float32(-jnp.inf) if is_max else jnp.float32(0.0)
  v = vals
  for s in (1, 2, 4, 8):
    vbuf[pl.ds(16, 16)] = v
    sv = _shifted(vbuf, iota, s)
    si = _shifted(ibuf, iota, s)
    contrib = jnp.where(si == ids, sv, neutral)
    v = jnp.maximum(v, contrib) if is_max else v + contrib
  return v


def _k2_body(bid_hbm, at_hbm, alpha_hbm, cnt_hbm, boff_hbm,
             bid_v, a_v, ibuf, vbuf, mloc, sloc, cloc, tmp, tmpc,
             red, credv, mful_v, sful_v, alpha_st, cnt_v, boff_v,
             mbuf_sh, cbuf_sh, mful_sh, sful_sh, cful_sh):
  c = lax.axis_index("c")
  s = lax.axis_index("s")
  iota = lax.iota(jnp.int32, 16)
  base = s * CHUNK

  pltpu.sync_copy(bid_hbm.at[pl.ds(base, CHUNK)], bid_v.at[pl.ds(0, CHUNK)])
  bid_v[pl.ds(CHUNK, 16)] = jnp.full((16,), _SENT, jnp.int32)
  for d in range(D):
    pltpu.sync_copy(at_hbm.at[pl.ds(d * NP + base, CHUNK)],
                    a_v.at[pl.ds(d * CHUNK, CHUNK)])

  ibuf[pl.ds(0, 16)] = jnp.full((16,), -1, jnp.int32)
  vbuf[pl.ds(0, 16)] = jnp.zeros((16,), jnp.float32)

  def _init(i, _):
    off = i * 16
    for d in range(D):
      mloc[d, pl.ds(off, 16)] = jnp.full((16,), -jnp.inf, jnp.float32)
      sloc[d, pl.ds(off, 16)] = jnp.zeros((16,), jnp.float32)
    cloc[pl.ds(off, 16)] = jnp.zeros((16,), jnp.float32)
    return 0
  lax.fori_loop(0, BP // 16, _init, 0)

  def _s1(v, _):
    b0 = v * 16
    ids = bid_v[pl.ds(b0, 16)]
    ids_nx = bid_v[pl.ds(b0 + 1, 16)]
    lastm = jnp.logical_or(ids != ids_nx, iota == 15)
    ibuf[pl.ds(16, 16)] = ids
    cv = _seg_combine(ids, jnp.ones((16,), jnp.float32), ibuf, vbuf, iota,
                      is_max=False)
    plsc.addupdate_scatter(cloc, [ids], cv, mask=lastm)
    for d in range(D):
      dsp = jnp.full((16,), d, jnp.int32)
      av = a_v[pl.ds(d * CHUNK + b0, 16)]
      mv = _seg_combine(ids, av, ibuf, vbuf, iota, is_max=True)
      cur = plsc.load_gather(mloc, [dsp, ids])
      plsc.store_scatter(mloc, [dsp, ids], jnp.maximum(cur, mv), mask=lastm)
    return 0
  lax.fori_loop(0, NV, _s1, 0)

  pltpu.sync_copy(mloc, mbuf_sh.at[s])
  pltpu.sync_copy(cloc, cbuf_sh.at[s])
  plsc.subcore_barrier()

  def _initred(i, _):
    off = i * 16
    for d in range(D):
      red[d, pl.ds(off, 16)] = jnp.full((16,), -jnp.inf, jnp.float32)
    credv[pl.ds(off, 16)] = jnp.zeros((16,), jnp.float32)
    return 0
  lax.fori_loop(0, 8, _initred, 0)
  win = s * 128
  for u in range(NT):
    pltpu.sync_copy(mbuf_sh.at[u], tmp)
    pltpu.sync_copy(cbuf_sh.at[u], tmpc)
    for d in range(D):
      for vv in range(8):
        o = vv * 16
        red[d, pl.ds(o, 16)] = jnp.maximum(
            red[d, pl.ds(o, 16)], tmp[d, pl.ds(win + o, 16)])
    for vv in range(8):
      o = vv * 16
      credv[pl.ds(o, 16)] = credv[pl.ds(o, 16)] + tmpc[pl.ds(win + o, 16)]
  pltpu.sync_copy(red, mful_sh.at[s])
  pltpu.sync_copy(credv, cful_sh.at[s])
  plsc.subcore_barrier()
  pltpu.sync_copy(mful_sh, mful_v)

  @pl.when(s == 0)
  def _():
    pltpu.sync_copy(cful_sh, cnt_hbm.at[c])

  @pl.when(jnp.logical_and(c == 0, s == 0))
  def _():
    pltpu.sync_copy(cful_sh, cnt_v)

    def _pfx(i, carry):
      q = lax.shift_right_logical(i, 3)
      o = jnp.bitwise_and(i, 7) * 16
      v = cnt_v[q, pl.ds(o, 16)]
      ex = plsc.cumsum(v) - v + carry
      boff_v[pl.ds(i * 16, 16)] = ex.astype(jnp.int32)
      return carry + jnp.sum(v)
    lax.fori_loop(0, BP // 16, _pfx, jnp.float32(0.0))
    pltpu.sync_copy(boff_v, boff_hbm)

  def _s2(v, _):
    b0 = v * 16
    ids = bid_v[pl.ds(b0, 16)]
    ids_nx = bid_v[pl.ds(b0 + 1, 16)]
    lastm = jnp.logical_or(ids != ids_nx, iota == 15)
    ibuf[pl.ds(16, 16)] = ids
    q = lax.shift_right_logical(ids, 7)
    r7 = jnp.bitwise_and(ids, 127)
    for d in range(D):
      dsp = jnp.full((16,), d, jnp.int32)
      m = plsc.load_gather(mful_v, [q, dsp, r7])
      ev = jnp.exp(a_v[pl.ds(d * CHUNK + b0, 16)] - m)
      ev = _seg_combine(ids, ev, ibuf, vbuf, iota, is_max=False)
      plsc.addupdate_scatter(sloc, [dsp, ids], ev, mask=lastm)
    return 0
  lax.fori_loop(0, NV, _s2, 0)

  pltpu.sync_copy(sloc, mbuf_sh.at[s])
  plsc.subcore_barrier()

  def _initred2(i, _):
    off = i * 16
    for d in range(D):
      red[d, pl.ds(off, 16)] = jnp.zeros((16,), jnp.float32)
    return 0
  lax.fori_loop(0, 8, _initred2, 0)
  for u in range(NT):
    pltpu.sync_copy(mbuf_sh.at[u], tmp)
    for d in range(D):
      for vv in range(8):
        o = vv * 16
        red[d, pl.ds(o, 16)] = red[d, pl.ds(o, 16)] + tmp[d, pl.ds(win + o, 16)]
  pltpu.sync_copy(red, sful_sh.at[s])
  plsc.subcore_barrier()
  pltpu.sync_copy(sful_sh, sful_v)

  coff = c * HALF

  def _s3(v, _):
    b0 = coff + v * 16
    ids = bid_v[pl.ds(b0, 16)]
    q = lax.shift_right_logical(ids, 7)
    r7 = jnp.bitwise_and(ids, 127)
    for d in range(D):
      dsp = jnp.full((16,), d, jnp.int32)
      m = plsc.load_gather(mful_v, [q, dsp, r7])
      sv = plsc.load_gather(sful_v, [q, dsp, r7])
      al = jnp.exp(a_v[pl.ds(d * CHUNK + b0, 16)] - m) / (sv + 1e-9)
      alpha_st[pl.ds(d * HALF + v * 16, 16)] = al
    return 0
  lax.fori_loop(0, NV // 2, _s3, 0)
  for d in range(D):
    pltpu.sync_copy(alpha_st.at[pl.ds(d * HALF, HALF)],
                    alpha_hbm.at[pl.ds(d * NP + base + coff, HALF)])


def _k2(bid_p, a_t_flat):
  mesh = plsc.VectorSubcoreMesh(core_axis_name="c", subcore_axis_name="s")
  f = pl.kernel(
      _k2_body,
      out_type=(
          jax.ShapeDtypeStruct((8 * NP,), jnp.float32),
          jax.ShapeDtypeStruct((2, NT, 128), jnp.float32),
          jax.ShapeDtypeStruct((BP,), jnp.int32),
      ),
      mesh=mesh,
      scratch_types=[
          pltpu.VMEM((CHUNK + 16,), jnp.int32),
          pltpu.VMEM((D * CHUNK,), jnp.float32),
          pltpu.VMEM((32,), jnp.int32),
          pltpu.VMEM((32,), jnp.float32),
          pltpu.VMEM((D, BP), jnp.float32),
          pltpu.VMEM((D, BP), jnp.float32),
          pltpu.VMEM((BP,), jnp.float32),
          pltpu.VMEM((D, BP), jnp.float32),
          pltpu.VMEM((BP,), jnp.float32),
          pltpu.VMEM((D, 128), jnp.float32),
          pltpu.VMEM((128,), jnp.float32),
          pltpu.VMEM((NT, D, 128), jnp.float32),
          pltpu.VMEM((NT, D, 128), jnp.float32),
          pltpu.VMEM((D * HALF,), jnp.float32),
          pltpu.VMEM((NT, 128), jnp.float32),
          pltpu.VMEM((BP,), jnp.int32),
          pltpu.VMEM_SHARED((NT, D, BP), jnp.float32),
          pltpu.VMEM_SHARED((NT, BP), jnp.float32),
          pltpu.VMEM_SHARED((NT, D, 128), jnp.float32),
          pltpu.VMEM_SHARED((NT, D, 128), jnp.float32),
          pltpu.VMEM_SHARED((NT, 128), jnp.float32),
      ],
      compiler_params=pltpu.CompilerParams(needs_layout_passes=False),
  )
  return f(bid_p, a_t_flat)


def _k3_body(hv_ref, al_ref, wa_ref, ba_ref, out_ref):
  h = hv_ref[...]
  al = jnp.transpose(al_ref[...], (1, 0))
  wa = wa_ref[...]
  ba = ba_ref[...]
  for d in range(D):
    p = jnp.dot(h, wa[d], preferred_element_type=jnp.float32) + ba[d]
    p = jnp.maximum(p, 0.0) * al[:, d:d + 1]
    out_ref[:, d * F:(d + 1) * F] = p


def _k3(hv, alpha_t, wa, ba):
  return pl.pallas_call(
      _k3_body,
      grid=(NBLK,),
      in_specs=[
          pl.BlockSpec((BLK, F), lambda i: (i, 0)),
          pl.BlockSpec((8, BLK), lambda i: (0, i)),
          pl.BlockSpec((D, F, F), lambda i: (0, 0, 0)),
          pl.BlockSpec((D, F), lambda i: (0, 0)),
      ],
      out_specs=pl.BlockSpec((BLK, D * F), lambda i: (i, 0)),
      out_shape=jax.ShapeDtypeStruct((NP, D * F), jnp.float32),
      compiler_params=pltpu.CompilerParams(
          dimension_semantics=("arbitrary",)),
  )(hv, alpha_t, wa, ba)


GPC = B // 32
ACCR = GPC + 2
AW = F + D * F


def _k4_body(bid_hbm, boff_hbm, hv_hbm, watt_hbm, pool_hbm,
             bo_v, bidv_r, locb, bh, bw, acc):
  c = lax.axis_index("c")
  s = lax.axis_index("s")
  iota = lax.iota(jnp.int32, 16)
  wid = c * NT + s
  g0 = pl.multiple_of(wid * GPC, GPC)

  pltpu.sync_copy(boff_hbm.at[pl.ds(g0, 48)], bo_v)
  start = jnp.min(plsc.load_gather(bo_v, [jnp.zeros((16,), jnp.int32)]))
  end = jnp.min(plsc.load_gather(bo_v, [jnp.full((16,), GPC, jnp.int32)]))
  start = pl.multiple_of(jnp.bitwise_and(start, ~15), 16)
  end = jnp.bitwise_and(end + 15, ~15)
  nch = lax.shift_right_logical(end - start, 4)

  def _zr(r, _):
    for k in range(AW // 16):
      acc[r, pl.ds(k * 16, 16)] = jnp.zeros((16,), jnp.float32)
    return 0
  lax.fori_loop(0, ACCR, _zr, 0)

  def _chunk(ch, _):
    r0 = pl.multiple_of(start + ch * 16, 16)
    pltpu.sync_copy(bid_hbm.at[pl.ds(r0, 16)], bidv_r)
    pltpu.sync_copy(hv_hbm.at[pl.ds(r0, 16)], bh)
    pltpu.sync_copy(watt_hbm.at[pl.ds(r0, 16)], bw)
    loc = bidv_r[pl.ds(0, 16)] - g0
    loc = jnp.where(
        jnp.logical_or(loc < 0, loc >= GPC), jnp.int32(GPC), loc)
    locb[pl.ds(0, 16)] = loc

    for h in (0, 1):
      inh = jnp.logical_and(iota >= 8 * h, iota < 8 * h + 8)
      mn = jnp.min(jnp.where(inh, loc, jnp.int32(BP)))
      mx = jnp.max(jnp.where(inh, loc, jnp.int32(-1)))

      @pl.when(mn == mx)
      def _fast():
        rowv = jnp.zeros((16,), jnp.int32) + mn
        for k in range(F // 16):
          ssum = bh[8 * h, pl.ds(k * 16, 16)]
          for j in range(8 * h + 1, 8 * h + 8):
            ssum = ssum + bh[j, pl.ds(k * 16, 16)]
          plsc.addupdate_scatter(acc, [rowv, iota + k * 16], ssum)
        for k in range(D * F // 16):
          ssum = bw[8 * h, pl.ds(k * 16, 16)]
          for j in range(8 * h + 1, 8 * h + 8):
            ssum = ssum + bw[j, pl.ds(k * 16, 16)]
          plsc.addupdate_scatter(acc, [rowv, iota + (F + k * 16)], ssum)

      @pl.when(mn != mx)
      def _slow():
        def _row(j, _):
          rowv = plsc.load_gather(locb, [jnp.zeros((16,), jnp.int32) + j])
          for k in range(F // 16):
            plsc.addupdate_scatter(
                acc, [rowv, iota + k * 16], bh[j, pl.ds(k * 16, 16)])
          for k in range(D * F // 16):
            plsc.addupdate_scatter(
                acc, [rowv, iota + (F + k * 16)], bw[j, pl.ds(k * 16, 16)])
          return 0
        lax.fori_loop(8 * h, 8 * h + 8, _row, 0)
    return 0
  lax.fori_loop(0, nch, _chunk, 0)

  pltpu.sync_copy(acc.at[pl.ds(0, GPC)], pool_hbm.at[pl.ds(g0, GPC)])


def _k4(bid_p, boffs, hv, watt):
  mesh = plsc.VectorSubcoreMesh(core_axis_name="c", subcore_axis_name="s")
  f = pl.kernel(
      _k4_body,
      out_type=jax.ShapeDtypeStruct((B, AW), jnp.float32),
      mesh=mesh,
      scratch_types=[
          pltpu.VMEM((48,), jnp.int32),
          pltpu.VMEM((16,), jnp.int32),
          pltpu.VMEM((16,), jnp.int32),
          pltpu.VMEM((16, F), jnp.float32),
          pltpu.VMEM((16, D * F), jnp.float32),
          pltpu.VMEM((ACCR, AW), jnp.float32),
      ],
      compiler_params=pltpu.CompilerParams(needs_layout_passes=False),
  )
  return f(bid_p, boffs, hv, watt)


def _k5_body(pool_ref, cnt_ref, gk_ref, grk_ref, gb_ref, out_ref):
  pool = pool_ref[...]
  hvsum = pool[:, :F]
  cnt = jnp.transpose(cnt_ref[...], (1, 0))
  gk = gk_ref[...]
  grk = grk_ref[...]
  gb = gb_ref[...]
  hm = hvsum / jnp.maximum(cnt, 1.0)
  for i in range(D):
    mm = pool[:, F + i * F:F + (i + 1) * F]
    mm = jnp.where(mm > 0, mm, jnp.exp(jnp.minimum(mm, 0.0)) - 1.0)
    mx = jnp.dot(mm, gk, preferred_element_type=jnp.float32) + gb[0]
    mh = jnp.dot(hm, grk, preferred_element_type=jnp.float32) + gb[1]
    z = jax.nn.sigmoid(mx[:, :F] + mh[:, :F])
    r = jax.nn.sigmoid(mx[:, F:2 * F] + mh[:, F:2 * F])
    hh = jnp.tanh(mx[:, 2 * F:] + r * mh[:, 2 * F:])
    hm = jnp.maximum(z * hm + (1.0 - z) * hh, 0.0)
  out_ref[...] = hm


def _k5(pool, cnt_row, gru_k, gru_rk, gru_b):
  return pl.pallas_call(
      _k5_body,
      in_specs=[
          pl.BlockSpec((B, AW), lambda: (0, 0)),
          pl.BlockSpec((1, B), lambda: (0, 0)),
          pl.BlockSpec((F, 3 * F), lambda: (0, 0)),
          pl.BlockSpec((F, 3 * F), lambda: (0, 0)),
          pl.BlockSpec((2, 3 * F), lambda: (0, 0)),
      ],
      out_specs=pl.BlockSpec((B, F), lambda: (0, 0)),
      out_shape=jax.ShapeDtypeStruct((B, F), jnp.float32),
  )(pool, cnt_row, gru_k, gru_rk, gru_b)


def kernel(count_nodes, nodes, batch_id, Wv, bv, Wa, ba, Wal, bal,
           gru_k, gru_rk, gru_b):
  del count_nodes, bal
  nodes_p = jnp.concatenate(
      [nodes, jnp.zeros((NP - N, F), jnp.float32)], axis=0)
  bid_p = jnp.concatenate(
      [batch_id.astype(jnp.int32), jnp.full((NP - N,), B, jnp.int32)])
  wal8 = jnp.zeros((8, F), jnp.float32).at[:D].set(Wal[:, F:, 0])
  bv2 = bv.reshape(1, F)

  hv, a_t = _k1(nodes_p, Wv, bv2, wal8)
  alpha_flat, cnt2, boffs = _k2(bid_p, a_t.reshape(-1))
  watt = _k3(hv, alpha_flat.reshape(8, NP), Wa, ba)
  pool = _k4(bid_p, boffs, hv, watt)
  cnt_row = cnt2.reshape(2, NT * 128)[0:1, :B]
  return _k5(pool, cnt_row, gru_k, gru_rk, gru_b)


# speedup vs baseline: 1.1772x; 1.1772x over previous
import functools

import jax
import jax.numpy as jnp
from jax import lax
from jax.experimental import pallas as pl
from jax.experimental.pallas import tpu as pltpu
from jax.experimental.pallas import tpu_sc as plsc

N = 50000
F = 256
B = 1024
D = 4

NP = 50176
BLK = 512
NBLK = NP // BLK
BP = 2048
BACC = 1152
NT = 16
CHUNK = NP // NT
NV = CHUNK // 16
HALF = CHUNK // 2
KR = 56
GPT = HALF // KR
IDXR = 32
RPT = BACC // NT

_SENT = BP - 1


def _k1_body(nodes_ref, wv_ref, bv_ref, wal_ref, hv_ref, at_ref):
  x = nodes_ref[...]
  h = jnp.dot(x, wv_ref[...], preferred_element_type=jnp.float32)
  h = jnp.maximum(h + bv_ref[...], 0.0)
  hv_ref[...] = h
  at_ref[...] = lax.dot_general(
      wal_ref[...], h, (((1,), (1,)), ((), ())),
      preferred_element_type=jnp.float32)


def _k1(nodes_p, wv, bv2, wal8):
  return pl.pallas_call(
      _k1_body,
      grid=(NBLK,),
      in_specs=[
          pl.BlockSpec((BLK, F), lambda i: (i, 0)),
          pl.BlockSpec((F, F), lambda i: (0, 0)),
          pl.BlockSpec((1, F), lambda i: (0, 0)),
          pl.BlockSpec((8, F), lambda i: (0, 0)),
      ],
      out_specs=[
          pl.BlockSpec((BLK, F), lambda i: (i, 0)),
          pl.BlockSpec((8, BLK), lambda i: (0, i)),
      ],
      out_shape=[
          jax.ShapeDtypeStruct((NP, F), jnp.float32),
          jax.ShapeDtypeStruct((8, NP), jnp.float32),
      ],
      compiler_params=pltpu.CompilerParams(
          dimension_semantics=("arbitrary",)),
  )(nodes_p, wv, bv2, wal8)


_IOTA = None


def _shifted(buf_ref, iota, shift):
  return plsc.load_gather(buf_ref, [iota + (16 - shift)])


def _seg_combine(ids, vals, ibuf, vbuf, iota, is_max):
  neutral = jnp.float32(-jnp.inf) if is_max else jnp.float32(0.0)
  v = vals
  for s in (1, 2, 4, 8):
    vbuf[pl.ds(16, 16)] = v
    sv = _shifted(vbuf, iota, s)
    si = _shifted(ibuf, iota, s)
    contrib = jnp.where(si == ids, sv, neutral)
    v = jnp.maximum(v, contrib) if is_max else v + contrib
  return v


def _k2_body(bid_hbm, at_hbm, alpha_hbm, cnt_hbm, boff_hbm,
             bid_v, a_v, ibuf, vbuf, mloc, sloc, cloc, tmp, tmpc,
             red, credv, mful_v, sful_v, alpha_st, cnt_v, boff_v,
             mbuf_sh, cbuf_sh, mful_sh, sful_sh, cful_sh):
  c = lax.axis_index("c")
  s = lax.axis_index("s")
  iota = lax.iota(jnp.int32, 16)
  base = s * CHUNK

  pltpu.sync_copy(bid_hbm.at[pl.ds(base, CHUNK)], bid_v.at[pl.ds(0, CHUNK)])
  bid_v[pl.ds(CHUNK, 16)] = jnp.full((16,), _SENT, jnp.int32)
  for d in range(D):
    pltpu.sync_copy(at_hbm.at[pl.ds(d * NP + base, CHUNK)],
                    a_v.at[pl.ds(d * CHUNK, CHUNK)])

  ibuf[pl.ds(0, 16)] = jnp.full((16,), -1, jnp.int32)
  vbuf[pl.ds(0, 16)] = jnp.zeros((16,), jnp.float32)

  def _init(i, _):
    off = i * 16
    for d in range(D):
      mloc[d, pl.ds(off, 16)] = jnp.full((16,), -jnp.inf, jnp.float32)
      sloc[d, pl.ds(off, 16)] = jnp.zeros((16,), jnp.float32)
    cloc[pl.ds(off, 16)] = jnp.zeros((16,), jnp.float32)
    return 0
  lax.fori_loop(0, BP // 16, _init, 0)

  def _s1(v, _):
    b0 = v * 16
    ids = bid_v[pl.ds(b0, 16)]
    ids_nx = bid_v[pl.ds(b0 + 1, 16)]
    lastm = jnp.logical_or(ids != ids_nx, iota == 15)
    ibuf[pl.ds(16, 16)] = ids
    cv = _seg_combine(ids, jnp.ones((16,), jnp.float32), ibuf, vbuf, iota,
                      is_max=False)
    plsc.addupdate_scatter(cloc, [ids], cv, mask=lastm)
    for d in range(D):
      dsp = jnp.full((16,), d, jnp.int32)
      av = a_v[pl.ds(d * CHUNK + b0, 16)]
      mv = _seg_combine(ids, av, ibuf, vbuf, iota, is_max=True)
      cur = plsc.load_gather(mloc, [dsp, ids])
      plsc.store_scatter(mloc, [dsp, ids], jnp.maximum(cur, mv), mask=lastm)
    return 0
  lax.fori_loop(0, NV, _s1, 0)

  pltpu.sync_copy(mloc, mbuf_sh.at[s])
  pltpu.sync_copy(cloc, cbuf_sh.at[s])
  plsc.subcore_barrier()

  def _initred(i, _):
    off = i * 16
    for d in range(D):
      red[d, pl.ds(off, 16)] = jnp.full((16,), -jnp.inf, jnp.float32)
    credv[pl.ds(off, 16)] = jnp.zeros((16,), jnp.float32)
    return 0
  lax.fori_loop(0, 8, _initred, 0)
  win = s * 128
  for u in range(NT):
    pltpu.sync_copy(mbuf_sh.at[u], tmp)
    pltpu.sync_copy(cbuf_sh.at[u], tmpc)
    for d in range(D):
      for vv in range(8):
        o = vv * 16
        red[d, pl.ds(o, 16)] = jnp.maximum(
            red[d, pl.ds(o, 16)], tmp[d, pl.ds(win + o, 16)])
    for vv in range(8):
      o = vv * 16
      credv[pl.ds(o, 16)] = credv[pl.ds(o, 16)] + tmpc[pl.ds(win + o, 16)]
  pltpu.sync_copy(red, mful_sh.at[s])
  pltpu.sync_copy(credv, cful_sh.at[s])
  plsc.subcore_barrier()
  pltpu.sync_copy(mful_sh, mful_v)

  @pl.when(s == 0)
  def _():
    pltpu.sync_copy(cful_sh, cnt_hbm.at[c])

  @pl.when(jnp.logical_and(c == 0, s == 0))
  def _():
    pltpu.sync_copy(cful_sh, cnt_v)

    def _pfx(i, carry):
      q = lax.shift_right_logical(i, 3)
      o = jnp.bitwise_and(i, 7) * 16
      v = cnt_v[q, pl.ds(o, 16)]
      ex = plsc.cumsum(v) - v + carry
      boff_v[pl.ds(i * 16, 16)] = ex.astype(jnp.int32)
      return carry + jnp.sum(v)
    lax.fori_loop(0, BP // 16, _pfx, jnp.float32(0.0))
    pltpu.sync_copy(boff_v, boff_hbm)

  def _s2(v, _):
    b0 = v * 16
    ids = bid_v[pl.ds(b0, 16)]
    ids_nx = bid_v[pl.ds(b0 + 1, 16)]
    lastm = jnp.logical_or(ids != ids_nx, iota == 15)
    ibuf[pl.ds(16, 16)] = ids
    q = lax.shift_right_logical(ids, 7)
    r7 = jnp.bitwise_and(ids, 127)
    for d in range(D):
      dsp = jnp.full((16,), d, jnp.int32)
      m = plsc.load_gather(mful_v, [q, dsp, r7])
      ev = jnp.exp(a_v[pl.ds(d * CHUNK + b0, 16)] - m)
      ev = _seg_combine(ids, ev, ibuf, vbuf, iota, is_max=False)
      plsc.addupdate_scatter(sloc, [dsp, ids], ev, mask=lastm)
    return 0
  lax.fori_loop(0, NV, _s2, 0)

  pltpu.sync_copy(sloc, mbuf_sh.at[s])
  plsc.subcore_barrier()

  def _initred2(i, _):
    off = i * 16
    for d in range(D):
      red[d, pl.ds(off, 16)] = jnp.zeros((16,), jnp.float32)
    return 0
  lax.fori_loop(0, 8, _initred2, 0)
  for u in range(NT):
    pltpu.sync_copy(mbuf_sh.at[u], tmp)
    for d in range(D):
      for vv in range(8):
        o = vv * 16
        red[d, pl.ds(o, 16)] = red[d, pl.ds(o, 16)] + tmp[d, pl.ds(win + o, 16)]
  pltpu.sync_copy(red, sful_sh.at[s])
  plsc.subcore_barrier()
  pltpu.sync_copy(sful_sh, sful_v)

  coff = c * HALF

  def _s3(v, _):
    b0 = coff + v * 16
    ids = bid_v[pl.ds(b0, 16)]
    q = lax.shift_right_logical(ids, 7)
    r7 = jnp.bitwise_and(ids, 127)
    for d in range(D):
      dsp = jnp.full((16,), d, jnp.int32)
      m = plsc.load_gather(mful_v, [q, dsp, r7])
      sv = plsc.load_gather(sful_v, [q, dsp, r7])
      al = jnp.exp(a_v[pl.ds(d * CHUNK + b0, 16)] - m) / (sv + 1e-9)
      alpha_st[pl.ds(d * HALF + v * 16, 16)] = al
    return 0
  lax.fori_loop(0, NV // 2, _s3, 0)
  for d in range(D):
    pltpu.sync_copy(alpha_st.at[pl.ds(d * HALF, HALF)],
                    alpha_hbm.at[pl.ds(d * NP + base + coff, HALF)])


def _k2(bid_p, a_t_flat):
  mesh = plsc.VectorSubcoreMesh(core_axis_name="c", subcore_axis_name="s")
  f = pl.kernel(
      _k2_body,
      out_type=(
          jax.ShapeDtypeStruct((8 * NP,), jnp.float32),
          jax.ShapeDtypeStruct((2, NT, 128), jnp.float32),
          jax.ShapeDtypeStruct((BP,), jnp.int32),
      ),
      mesh=mesh,
      scratch_types=[
          pltpu.VMEM((CHUNK + 16,), jnp.int32),
          pltpu.VMEM((D * CHUNK,), jnp.float32),
          pltpu.VMEM((32,), jnp.int32),
          pltpu.VMEM((32,), jnp.float32),
          pltpu.VMEM((D, BP), jnp.float32),
          pltpu.VMEM((D, BP), jnp.float32),
          pltpu.VMEM((BP,), jnp.float32),
          pltpu.VMEM((D, BP), jnp.float32),
          pltpu.VMEM((BP,), jnp.float32),
          pltpu.VMEM((D, 128), jnp.float32),
          pltpu.VMEM((128,), jnp.float32),
          pltpu.VMEM((NT, D, 128), jnp.float32),
          pltpu.VMEM((NT, D, 128), jnp.float32),
          pltpu.VMEM((D * HALF,), jnp.float32),
          pltpu.VMEM((NT, 128), jnp.float32),
          pltpu.VMEM((BP,), jnp.int32),
          pltpu.VMEM_SHARED((NT, D, BP), jnp.float32),
          pltpu.VMEM_SHARED((NT, BP), jnp.float32),
          pltpu.VMEM_SHARED((NT, D, 128), jnp.float32),
          pltpu.VMEM_SHARED((NT, D, 128), jnp.float32),
          pltpu.VMEM_SHARED((NT, 128), jnp.float32),
      ],
      compiler_params=pltpu.CompilerParams(needs_layout_passes=False),
  )
  return f(bid_p, a_t_flat)


def _k3_body(hv_ref, al_ref, wa_ref, ba_ref, out_ref):
  h = hv_ref[...]
  al = jnp.transpose(al_ref[...], (1, 0))
  wa = wa_ref[...]
  ba = ba_ref[...]
  for d in range(D):
    p = jnp.dot(h, wa[d], preferred_element_type=jnp.float32) + ba[d]
    p = jnp.maximum(p, 0.0) * al[:, d:d + 1]
    out_ref[:, d * F:(d + 1) * F] = p


def _k3(hv, alpha_t, wa, ba):
  return pl.pallas_call(
      _k3_body,
      grid=(NBLK,),
      in_specs=[
          pl.BlockSpec((BLK, F), lambda i: (i, 0)),
          pl.BlockSpec((8, BLK), lambda i: (0, i)),
          pl.BlockSpec((D, F, F), lambda i: (0, 0, 0)),
          pl.BlockSpec((D, F), lambda i: (0, 0)),
      ],
      out_specs=pl.BlockSpec((BLK, D * F), lambda i: (i, 0)),
      out_shape=jax.ShapeDtypeStruct((NP, D * F), jnp.float32),
      compiler_params=pltpu.CompilerParams(
          dimension_semantics=("arbitrary",)),
  )(hv, alpha_t, wa, ba)


GPC = B // 32
ACCR = GPC + 2
AW = F + D * F


def _k4_body(bid_hbm, boff_hbm, hv_hbm, watt_hbm, pool_hbm,
             bo_v, bidv_r, locb, bh, bw, acc, sem1, sem2, sem3):
  c = lax.axis_index("c")
  s = lax.axis_index("s")
  iota = lax.iota(jnp.int32, 16)
  wid = c * NT + s
  g0 = pl.multiple_of(wid * GPC, GPC)

  pltpu.sync_copy(boff_hbm.at[pl.ds(g0, 48)], bo_v)
  start = jnp.min(plsc.load_gather(bo_v, [jnp.zeros((16,), jnp.int32)]))
  end = jnp.min(plsc.load_gather(bo_v, [jnp.full((16,), GPC, jnp.int32)]))
  start = pl.multiple_of(jnp.bitwise_and(start, ~31), 32)
  end = jnp.bitwise_and(end + 31, ~31)
  nch = lax.shift_right_logical(end - start, 5)

  def _zr(r, _):
    for k in range(AW // 16):
      acc[r, pl.ds(k * 16, 16)] = jnp.zeros((16,), jnp.float32)
    return 0
  lax.fori_loop(0, ACCR, _zr, 0)

  def _chunk(ch, _):
    r0 = pl.multiple_of(start + ch * 32, 32)
    d1 = pltpu.async_copy(bid_hbm.at[pl.ds(r0, 32)], bidv_r, sem1)
    d2 = pltpu.async_copy(hv_hbm.at[pl.ds(r0, 32)], bh, sem2)
    d3 = pltpu.async_copy(watt_hbm.at[pl.ds(r0, 32)], bw, sem3)
    d1.wait()
    for hh in (0, 1):
      loc = bidv_r[pl.ds(16 * hh, 16)] - g0
      loc = jnp.where(
          jnp.logical_or(loc < 0, loc >= GPC), jnp.int32(GPC), loc)
      locb[pl.ds(16 * hh, 16)] = loc
    d2.wait()
    d3.wait()

    def _row(j, _):
      rowv = plsc.load_gather(locb, [jnp.zeros((16,), jnp.int32) + j])
      for k in range(F // 16):
        plsc.addupdate_scatter(
            acc, [rowv, iota + k * 16], bh[j, pl.ds(k * 16, 16)])
      for k in range(D * F // 16):
        plsc.addupdate_scatter(
            acc, [rowv, iota + (F + k * 16)], bw[j, pl.ds(k * 16, 16)])
      return 0
    lax.fori_loop(0, 32, _row, 0)
    return 0
  lax.fori_loop(0, nch, _chunk, 0)

  pltpu.sync_copy(acc.at[pl.ds(0, GPC)], pool_hbm.at[pl.ds(g0, GPC)])


def _k4(bid_p, boffs, hv, watt):
  mesh = plsc.VectorSubcoreMesh(core_axis_name="c", subcore_axis_name="s")
  f = pl.kernel(
      _k4_body,
      out_type=jax.ShapeDtypeStruct((B, AW), jnp.float32),
      mesh=mesh,
      scratch_types=[
          pltpu.VMEM((48,), jnp.int32),
          pltpu.VMEM((32,), jnp.int32),
          pltpu.VMEM((32,), jnp.int32),
          pltpu.VMEM((32, F), jnp.float32),
          pltpu.VMEM((32, D * F), jnp.float32),
          pltpu.VMEM((ACCR, AW), jnp.float32),
          pltpu.SemaphoreType.DMA,
          pltpu.SemaphoreType.DMA,
          pltpu.SemaphoreType.DMA,
      ],
      compiler_params=pltpu.CompilerParams(needs_layout_passes=False),
  )
  return f(bid_p, boffs, hv, watt)


def _k5_body(pool_ref, cnt_ref, gk_ref, grk_ref, gb_ref, out_ref):
  pool = pool_ref[...]
  hvsum = pool[:, :F]
  cnt = jnp.transpose(cnt_ref[...], (1, 0))
  gk = gk_ref[...]
  grk = grk_ref[...]
  gb = gb_ref[...]
  hm = hvsum / jnp.maximum(cnt, 1.0)
  for i in range(D):
    mm = pool[:, F + i * F:F + (i + 1) * F]
    mm = jnp.where(mm > 0, mm, jnp.exp(jnp.minimum(mm, 0.0)) - 1.0)
    mx = jnp.dot(mm, gk, preferred_element_type=jnp.float32) + gb[0]
    mh = jnp.dot(hm, grk, preferred_element_type=jnp.float32) + gb[1]
    z = jax.nn.sigmoid(mx[:, :F] + mh[:, :F])
    r = jax.nn.sigmoid(mx[:, F:2 * F] + mh[:, F:2 * F])
    hh = jnp.tanh(mx[:, 2 * F:] + r * mh[:, 2 * F:])
    hm = jnp.maximum(z * hm + (1.0 - z) * hh, 0.0)
  out_ref[...] = hm


def _k5(pool, cnt_row, gru_k, gru_rk, gru_b):
  return pl.pallas_call(
      _k5_body,
      in_specs=[
          pl.BlockSpec((B, AW), lambda: (0, 0)),
          pl.BlockSpec((1, B), lambda: (0, 0)),
          pl.BlockSpec((F, 3 * F), lambda: (0, 0)),
          pl.BlockSpec((F, 3 * F), lambda: (0, 0)),
          pl.BlockSpec((2, 3 * F), lambda: (0, 0)),
      ],
      out_specs=pl.BlockSpec((B, F), lambda: (0, 0)),
      out_shape=jax.ShapeDtypeStruct((B, F), jnp.float32),
  )(pool, cnt_row, gru_k, gru_rk, gru_b)


def kernel(count_nodes, nodes, batch_id, Wv, bv, Wa, ba, Wal, bal,
           gru_k, gru_rk, gru_b):
  del count_nodes, bal
  nodes_p = jnp.concatenate(
      [nodes, jnp.zeros((NP - N, F), jnp.float32)], axis=0)
  bid_p = jnp.concatenate(
      [batch_id.astype(jnp.int32), jnp.full((NP - N,), B, jnp.int32)])
  wal8 = jnp.zeros((8, F), jnp.float32).at[:D].set(Wal[:, F:, 0])
  bv2 = bv.reshape(1, F)

  hv, a_t = _k1(nodes_p, Wv, bv2, wal8)
  alpha_flat, cnt2, boffs = _k2(bid_p, a_t.reshape(-1))
  watt = _k3(hv, alpha_flat.reshape(8, NP), Wa, ba)
  pool = _k4(bid_p, boffs, hv, watt)
  cnt_row = cnt2.reshape(2, NT * 128)[0:1, :B]
  return _k5(pool, cnt_row, gru_k, gru_rk, gru_b)


# speedup vs baseline: 1.1935x; 1.0138x over previous
import functools

import jax
import jax.numpy as jnp
from jax import lax
from jax.experimental import pallas as pl
from jax.experimental.pallas import tpu as pltpu
from jax.experimental.pallas import tpu_sc as plsc

N = 50000
F = 256
B = 1024
D = 4

NP = 50176
BLK = 512
NBLK = NP // BLK
BP = 2048
BACC = 1152
NT = 16
CHUNK = NP // NT
NV = CHUNK // 16
HALF = CHUNK // 2
KR = 56
GPT = HALF // KR
IDXR = 32
RPT = BACC // NT

_SENT = BP - 1


def _k1_body(nodes_ref, wv_ref, bv_ref, wal_ref, hv_ref, at_ref):
  x = nodes_ref[...]
  h = jnp.dot(x, wv_ref[...], preferred_element_type=jnp.float32)
  h = jnp.maximum(h + bv_ref[...], 0.0)
  hv_ref[...] = h
  at_ref[...] = lax.dot_general(
      wal_ref[...], h, (((1,), (1,)), ((), ())),
      preferred_element_type=jnp.float32)


def _k1(nodes_p, wv, bv2, wal8):
  return pl.pallas_call(
      _k1_body,
      grid=(NBLK,),
      in_specs=[
          pl.BlockSpec((BLK, F), lambda i: (i, 0)),
          pl.BlockSpec((F, F), lambda i: (0, 0)),
          pl.BlockSpec((1, F), lambda i: (0, 0)),
          pl.BlockSpec((8, F), lambda i: (0, 0)),
      ],
      out_specs=[
          pl.BlockSpec((BLK, F), lambda i: (i, 0)),
          pl.BlockSpec((8, BLK), lambda i: (0, i)),
      ],
      out_shape=[
          jax.ShapeDtypeStruct((NP, F), jnp.float32),
          jax.ShapeDtypeStruct((8, NP), jnp.float32),
      ],
      compiler_params=pltpu.CompilerParams(
          dimension_semantics=("arbitrary",)),
  )(nodes_p, wv, bv2, wal8)


_IOTA = None


def _shifted(buf_ref, iota, shift):
  return plsc.load_gather(buf_ref, [iota + (16 - shift)])


def _seg_combine(ids, vals, ibuf, vbuf, iota, is_max):
  neutral = jnp.float32(-jnp.inf) if is_max else jnp.float32(0.0)
  v = vals
  for s in (1, 2, 4, 8):
    vbuf[pl.ds(16, 16)] = v
    sv = _shifted(vbuf, iota, s)
    si = _shifted(ibuf, iota, s)
    contrib = jnp.where(si == ids, sv, neutral)
    v = jnp.maximum(v, contrib) if is_max else v + contrib
  return v


def _k2_body(bid_hbm, at_hbm, alpha_hbm, cnt_hbm, boff_hbm,
             bid_v, a_v, ibuf, vbuf, mloc, sloc, cloc, tmp, tmpc,
             red, credv, mful_v, sful_v, alpha_st, cnt_v, boff_v,
             mbuf_sh, cbuf_sh, mful_sh, sful_sh, cful_sh):
  c = lax.axis_index("c")
  s = lax.axis_index("s")
  iota = lax.iota(jnp.int32, 16)
  base = s * CHUNK

  pltpu.sync_copy(bid_hbm.at[pl.ds(base, CHUNK)], bid_v.at[pl.ds(0, CHUNK)])
  bid_v[pl.ds(CHUNK, 16)] = jnp.full((16,), _SENT, jnp.int32)
  for d in range(D):
    pltpu.sync_copy(at_hbm.at[pl.ds(d * NP + base, CHUNK)],
                    a_v.at[pl.ds(d * CHUNK, CHUNK)])

  ibuf[pl.ds(0, 16)] = jnp.full((16,), -1, jnp.int32)
  vbuf[pl.ds(0, 16)] = jnp.zeros((16,), jnp.float32)

  def _init(i, _):
    off = i * 16
    for d in range(D):
      mloc[d, pl.ds(off, 16)] = jnp.full((16,), -jnp.inf, jnp.float32)
      sloc[d, pl.ds(off, 16)] = jnp.zeros((16,), jnp.float32)
    cloc[pl.ds(off, 16)] = jnp.zeros((16,), jnp.float32)
    return 0
  lax.fori_loop(0, BP // 16, _init, 0)

  def _s1(v, _):
    b0 = v * 16
    ids = bid_v[pl.ds(b0, 16)]
    ids_nx = bid_v[pl.ds(b0 + 1, 16)]
    lastm = jnp.logical_or(ids != ids_nx, iota == 15)
    ibuf[pl.ds(16, 16)] = ids
    cv = _seg_combine(ids, jnp.ones((16,), jnp.float32), ibuf, vbuf, iota,
                      is_max=False)
    plsc.addupdate_scatter(cloc, [ids], cv, mask=lastm)
    for d in range(D):
      dsp = jnp.full((16,), d, jnp.int32)
      av = a_v[pl.ds(d * CHUNK + b0, 16)]
      mv = _seg_combine(ids, av, ibuf, vbuf, iota, is_max=True)
      cur = plsc.load_gather(mloc, [dsp, ids])
      plsc.store_scatter(mloc, [dsp, ids], jnp.maximum(cur, mv), mask=lastm)
    return 0
  lax.fori_loop(0, NV, _s1, 0)

  pltpu.sync_copy(mloc, mbuf_sh.at[s])
  pltpu.sync_copy(cloc, cbuf_sh.at[s])
  plsc.subcore_barrier()

  def _initred(i, _):
    off = i * 16
    for d in range(D):
      red[d, pl.ds(off, 16)] = jnp.full((16,), -jnp.inf, jnp.float32)
    credv[pl.ds(off, 16)] = jnp.zeros((16,), jnp.float32)
    return 0
  lax.fori_loop(0, 8, _initred, 0)
  win = s * 128
  for u in range(NT):
    pltpu.sync_copy(mbuf_sh.at[u], tmp)
    pltpu.sync_copy(cbuf_sh.at[u], tmpc)
    for d in range(D):
      for vv in range(8):
        o = vv * 16
        red[d, pl.ds(o, 16)] = jnp.maximum(
            red[d, pl.ds(o, 16)], tmp[d, pl.ds(win + o, 16)])
    for vv in range(8):
      o = vv * 16
      credv[pl.ds(o, 16)] = credv[pl.ds(o, 16)] + tmpc[pl.ds(win + o, 16)]
  pltpu.sync_copy(red, mful_sh.at[s])
  pltpu.sync_copy(credv, cful_sh.at[s])
  plsc.subcore_barrier()
  pltpu.sync_copy(mful_sh, mful_v)

  @pl.when(s == 0)
  def _():
    pltpu.sync_copy(cful_sh, cnt_hbm.at[c])

  @pl.when(jnp.logical_and(c == 0, s == 0))
  def _():
    pltpu.sync_copy(cful_sh, cnt_v)

    def _pfx(i, carry):
      q = lax.shift_right_logical(i, 3)
      o = jnp.bitwise_and(i, 7) * 16
      v = cnt_v[q, pl.ds(o, 16)]
      ex = plsc.cumsum(v) - v + carry
      boff_v[pl.ds(i * 16, 16)] = ex.astype(jnp.int32)
      return carry + jnp.sum(v)
    lax.fori_loop(0, BP // 16, _pfx, jnp.float32(0.0))
    pltpu.sync_copy(boff_v, boff_hbm)

  def _s2(v, _):
    b0 = v * 16
    ids = bid_v[pl.ds(b0, 16)]
    ids_nx = bid_v[pl.ds(b0 + 1, 16)]
    lastm = jnp.logical_or(ids != ids_nx, iota == 15)
    ibuf[pl.ds(16, 16)] = ids
    q = lax.shift_right_logical(ids, 7)
    r7 = jnp.bitwise_and(ids, 127)
    for d in range(D):
      dsp = jnp.full((16,), d, jnp.int32)
      m = plsc.load_gather(mful_v, [q, dsp, r7])
      ev = jnp.exp(a_v[pl.ds(d * CHUNK + b0, 16)] - m)
      ev = _seg_combine(ids, ev, ibuf, vbuf, iota, is_max=False)
      plsc.addupdate_scatter(sloc, [dsp, ids], ev, mask=lastm)
    return 0
  lax.fori_loop(0, NV, _s2, 0)

  pltpu.sync_copy(sloc, mbuf_sh.at[s])
  plsc.subcore_barrier()

  def _initred2(i, _):
    off = i * 16
    for d in range(D):
      red[d, pl.ds(off, 16)] = jnp.zeros((16,), jnp.float32)
    return 0
  lax.fori_loop(0, 8, _initred2, 0)
  for u in range(NT):
    pltpu.sync_copy(mbuf_sh.at[u], tmp)
    for d in range(D):
      for vv in range(8):
        o = vv * 16
        red[d, pl.ds(o, 16)] = red[d, pl.ds(o, 16)] + tmp[d, pl.ds(win + o, 16)]
  pltpu.sync_copy(red, sful_sh.at[s])
  plsc.subcore_barrier()
  pltpu.sync_copy(sful_sh, sful_v)

  coff = c * HALF

  def _s3(v, _):
    b0 = coff + v * 16
    ids = bid_v[pl.ds(b0, 16)]
    q = lax.shift_right_logical(ids, 7)
    r7 = jnp.bitwise_and(ids, 127)
    for d in range(D):
      dsp = jnp.full((16,), d, jnp.int32)
      m = plsc.load_gather(mful_v, [q, dsp, r7])
      sv = plsc.load_gather(sful_v, [q, dsp, r7])
      al = jnp.exp(a_v[pl.ds(d * CHUNK + b0, 16)] - m) / (sv + 1e-9)
      alpha_st[pl.ds(d * HALF + v * 16, 16)] = al
    return 0
  lax.fori_loop(0, NV // 2, _s3, 0)
  for d in range(D):
    pltpu.sync_copy(alpha_st.at[pl.ds(d * HALF, HALF)],
                    alpha_hbm.at[pl.ds(d * NP + base + coff, HALF)])


def _k2(bid_p, a_t_flat):
  mesh = plsc.VectorSubcoreMesh(core_axis_name="c", subcore_axis_name="s")
  f = pl.kernel(
      _k2_body,
      out_type=(
          jax.ShapeDtypeStruct((8 * NP,), jnp.float32),
          jax.ShapeDtypeStruct((2, NT, 128), jnp.float32),
          jax.ShapeDtypeStruct((BP,), jnp.int32),
      ),
      mesh=mesh,
      scratch_types=[
          pltpu.VMEM((CHUNK + 16,), jnp.int32),
          pltpu.VMEM((D * CHUNK,), jnp.float32),
          pltpu.VMEM((32,), jnp.int32),
          pltpu.VMEM((32,), jnp.float32),
          pltpu.VMEM((D, BP), jnp.float32),
          pltpu.VMEM((D, BP), jnp.float32),
          pltpu.VMEM((BP,), jnp.float32),
          pltpu.VMEM((D, BP), jnp.float32),
          pltpu.VMEM((BP,), jnp.float32),
          pltpu.VMEM((D, 128), jnp.float32),
          pltpu.VMEM((128,), jnp.float32),
          pltpu.VMEM((NT, D, 128), jnp.float32),
          pltpu.VMEM((NT, D, 128), jnp.float32),
          pltpu.VMEM((D * HALF,), jnp.float32),
          pltpu.VMEM((NT, 128), jnp.float32),
          pltpu.VMEM((BP,), jnp.int32),
          pltpu.VMEM_SHARED((NT, D, BP), jnp.float32),
          pltpu.VMEM_SHARED((NT, BP), jnp.float32),
          pltpu.VMEM_SHARED((NT, D, 128), jnp.float32),
          pltpu.VMEM_SHARED((NT, D, 128), jnp.float32),
          pltpu.VMEM_SHARED((NT, 128), jnp.float32),
      ],
      compiler_params=pltpu.CompilerParams(needs_layout_passes=False),
  )
  return f(bid_p, a_t_flat)


def _k3_body(hv_ref, al_ref, wa_ref, ba_ref, out_ref):
  h = hv_ref[...]
  al = jnp.transpose(al_ref[...], (1, 0))
  wa = wa_ref[...]
  ba = ba_ref[...]
  for d in range(D):
    p = jnp.dot(h, wa[d], preferred_element_type=jnp.float32) + ba[d]
    p = jnp.maximum(p, 0.0) * al[:, d:d + 1]
    out_ref[:, d * F:(d + 1) * F] = p


def _k3(hv, alpha_t, wa, ba):
  return pl.pallas_call(
      _k3_body,
      grid=(NBLK,),
      in_specs=[
          pl.BlockSpec((BLK, F), lambda i: (i, 0)),
          pl.BlockSpec((8, BLK), lambda i: (0, i)),
          pl.BlockSpec((D, F, F), lambda i: (0, 0, 0)),
          pl.BlockSpec((D, F), lambda i: (0, 0)),
      ],
      out_specs=pl.BlockSpec((BLK, D * F), lambda i: (i, 0)),
      out_shape=jax.ShapeDtypeStruct((NP, D * F), jnp.float32),
      compiler_params=pltpu.CompilerParams(
          dimension_semantics=("arbitrary",)),
  )(hv, alpha_t, wa, ba)


GPC = B // 32
ACCR = GPC + 1
AW = F + D * F


def _k4_body(bid_hbm, boff_hbm, hv_hbm, watt_hbm, pool_hbm,
             bo_v, bidv_r, locb, bh0, bw0, bh1, bw1, acc,
             semb0, semh0, semw0, semb1, semh1, semw1):
  c = lax.axis_index("c")
  s = lax.axis_index("s")
  iota = lax.iota(jnp.int32, 16)
  wid = c * NT + s
  g0 = pl.multiple_of(wid * GPC, GPC)

  pltpu.sync_copy(boff_hbm.at[pl.ds(g0, 40)], bo_v)
  start = jnp.min(plsc.load_gather(bo_v, [jnp.zeros((16,), jnp.int32)]))
  end = jnp.min(plsc.load_gather(bo_v, [jnp.full((16,), GPC, jnp.int32)]))
  start = pl.multiple_of(jnp.bitwise_and(start, ~7), 8)
  end = jnp.bitwise_and(end + 7, ~7)
  nch = (end - start + 23) // 24
  nch2 = lax.shift_right_logical(nch + 1, 1)

  def _zr(r, _):
    for k in range(AW // 16):
      acc[r, pl.ds(k * 16, 16)] = jnp.zeros((16,), jnp.float32)
    return 0
  lax.fori_loop(0, ACCR, _zr, 0)

  bufs = ((bidv_r.at[pl.ds(0, 24)], bh0, bw0, semb0, semh0, semw0),
          (bidv_r.at[pl.ds(24, 24)], bh1, bw1, semb1, semh1, semw1))

  def _issue(i, b):
    bb, hb, wb, sb, sh, sw = bufs[b]
    r0 = pl.multiple_of(start + i * 24, 8)
    pltpu.async_copy(bid_hbm.at[pl.ds(r0, 24)], bb, sb)
    pltpu.async_copy(hv_hbm.at[pl.ds(r0, 24)], hb, sh)
    pltpu.async_copy(watt_hbm.at[pl.ds(r0, 24)], wb, sw)

  def _wait(b):
    bb, hb, wb, sb, sh, sw = bufs[b]
    pltpu.make_async_copy(bid_hbm.at[pl.ds(0, 24)], bb, sb).wait()
    pltpu.make_async_copy(hv_hbm.at[pl.ds(0, 24)], hb, sh).wait()
    pltpu.make_async_copy(watt_hbm.at[pl.ds(0, 24)], wb, sw).wait()

  def _compute(b):
    bb, hb, wb, _, _, _ = bufs[b]
    for hh in (0, 1):
      loc = bb[pl.ds(8 * hh, 16)] - g0
      loc = jnp.where(
          jnp.logical_or(loc < 0, loc >= GPC), jnp.int32(GPC), loc)
      locb[pl.ds(8 * hh, 16)] = loc

    def _row(j, _):
      rowv = plsc.load_gather(locb, [jnp.zeros((16,), jnp.int32) + j])
      for k in range(F // 16):
        plsc.addupdate_scatter(
            acc, [rowv, iota + k * 16], hb[j, pl.ds(k * 16, 16)])
      for k in range(D * F // 16):
        plsc.addupdate_scatter(
            acc, [rowv, iota + (F + k * 16)], wb[j, pl.ds(k * 16, 16)])
      return 0
    lax.fori_loop(0, 24, _row, 0)

  _issue(0, 0)

  def _pair(g, _):
    i = g * 2
    _wait(0)
    _issue(i + 1, 1)
    _compute(0)
    _wait(1)
    _issue(i + 2, 0)
    _compute(1)
    return 0
  lax.fori_loop(0, nch2, _pair, 0)
  _wait(0)

  pltpu.sync_copy(acc.at[pl.ds(0, GPC)], pool_hbm.at[pl.ds(g0, GPC)])


def _k4(bid_p, boffs, hv, watt):
  mesh = plsc.VectorSubcoreMesh(core_axis_name="c", subcore_axis_name="s")
  f = pl.kernel(
      _k4_body,
      out_type=jax.ShapeDtypeStruct((B, AW), jnp.float32),
      mesh=mesh,
      scratch_types=[
          pltpu.VMEM((40,), jnp.int32),
          pltpu.VMEM((48,), jnp.int32),
          pltpu.VMEM((32,), jnp.int32),
          pltpu.VMEM((24, F), jnp.float32),
          pltpu.VMEM((24, D * F), jnp.float32),
          pltpu.VMEM((24, F), jnp.float32),
          pltpu.VMEM((24, D * F), jnp.float32),
          pltpu.VMEM((ACCR, AW), jnp.float32),
          pltpu.SemaphoreType.DMA,
          pltpu.SemaphoreType.DMA,
          pltpu.SemaphoreType.DMA,
          pltpu.SemaphoreType.DMA,
          pltpu.SemaphoreType.DMA,
          pltpu.SemaphoreType.DMA,
      ],
      compiler_params=pltpu.CompilerParams(needs_layout_passes=False),
  )
  return f(bid_p, boffs, hv, watt)


def _k5_body(pool_ref, cnt_ref, gk_ref, grk_ref, gb_ref, out_ref):
  pool = pool_ref[...]
  hvsum = pool[:, :F]
  cnt = jnp.transpose(cnt_ref[...], (1, 0))
  gk = gk_ref[...]
  grk = grk_ref[...]
  gb = gb_ref[...]
  hm = hvsum / jnp.maximum(cnt, 1.0)
  for i in range(D):
    mm = pool[:, F + i * F:F + (i + 1) * F]
    mm = jnp.where(mm > 0, mm, jnp.exp(jnp.minimum(mm, 0.0)) - 1.0)
    mx = jnp.dot(mm, gk, preferred_element_type=jnp.float32) + gb[0]
    mh = jnp.dot(hm, grk, preferred_element_type=jnp.float32) + gb[1]
    z = jax.nn.sigmoid(mx[:, :F] + mh[:, :F])
    r = jax.nn.sigmoid(mx[:, F:2 * F] + mh[:, F:2 * F])
    hh = jnp.tanh(mx[:, 2 * F:] + r * mh[:, 2 * F:])
    hm = jnp.maximum(z * hm + (1.0 - z) * hh, 0.0)
  out_ref[...] = hm


def _k5(pool, cnt_row, gru_k, gru_rk, gru_b):
  return pl.pallas_call(
      _k5_body,
      in_specs=[
          pl.BlockSpec((B, AW), lambda: (0, 0)),
          pl.BlockSpec((1, B), lambda: (0, 0)),
          pl.BlockSpec((F, 3 * F), lambda: (0, 0)),
          pl.BlockSpec((F, 3 * F), lambda: (0, 0)),
          pl.BlockSpec((2, 3 * F), lambda: (0, 0)),
      ],
      out_specs=pl.BlockSpec((B, F), lambda: (0, 0)),
      out_shape=jax.ShapeDtypeStruct((B, F), jnp.float32),
  )(pool, cnt_row, gru_k, gru_rk, gru_b)


def kernel(count_nodes, nodes, batch_id, Wv, bv, Wa, ba, Wal, bal,
           gru_k, gru_rk, gru_b):
  del count_nodes, bal
  nodes_p = jnp.concatenate(
      [nodes, jnp.zeros((NP - N, F), jnp.float32)], axis=0)
  bid_p = jnp.concatenate(
      [batch_id.astype(jnp.int32), jnp.full((NP - N,), B, jnp.int32)])
  wal8 = jnp.zeros((8, F), jnp.float32).at[:D].set(Wal[:, F:, 0])
  bv2 = bv.reshape(1, F)

  hv, a_t = _k1(nodes_p, Wv, bv2, wal8)
  alpha_flat, cnt2, boffs = _k2(bid_p, a_t.reshape(-1))
  watt = _k3(hv, alpha_flat.reshape(8, NP), Wa, ba)
  pool = _k4(bid_p, boffs, hv, watt)
  cnt_row = cnt2.reshape(2, NT * 128)[0:1, :B]
  return _k5(pool, cnt_row, gru_k, gru_rk, gru_b)


# speedup vs baseline: 1.1940x; 1.0004x over previous
import functools

import jax
import jax.numpy as jnp
from jax import lax
from jax.experimental import pallas as pl
from jax.experimental.pallas import tpu as pltpu
from jax.experimental.pallas import tpu_sc as plsc

N = 50000
F = 256
B = 1024
D = 4

NP = 50176
BLK = 512
NBLK = NP // BLK
BP = 2048
BACC = 1152
NT = 16
CHUNK = NP // NT
NV = CHUNK // 16
HALF = CHUNK // 2
KR = 56
GPT = HALF // KR
IDXR = 32
RPT = BACC // NT

_SENT = BP - 1


def _k1_body(nodes_ref, wv_ref, bv_ref, wal_ref, hv_ref, at_ref):
  x = nodes_ref[...].astype(jnp.bfloat16)
  h = jnp.dot(x, wv_ref[...].astype(jnp.bfloat16),
              preferred_element_type=jnp.float32)
  h = jnp.maximum(h + bv_ref[...], 0.0)
  hv_ref[...] = h
  at_ref[...] = lax.dot_general(
      wal_ref[...], h, (((1,), (1,)), ((), ())),
      preferred_element_type=jnp.float32)


def _k1(nodes_p, wv, bv2, wal8):
  return pl.pallas_call(
      _k1_body,
      grid=(NBLK,),
      in_specs=[
          pl.BlockSpec((BLK, F), lambda i: (i, 0)),
          pl.BlockSpec((F, F), lambda i: (0, 0)),
          pl.BlockSpec((1, F), lambda i: (0, 0)),
          pl.BlockSpec((8, F), lambda i: (0, 0)),
      ],
      out_specs=[
          pl.BlockSpec((BLK, F), lambda i: (i, 0)),
          pl.BlockSpec((8, BLK), lambda i: (0, i)),
      ],
      out_shape=[
          jax.ShapeDtypeStruct((NP, F), jnp.float32),
          jax.ShapeDtypeStruct((8, NP), jnp.float32),
      ],
      compiler_params=pltpu.CompilerParams(
          dimension_semantics=("arbitrary",)),
  )(nodes_p, wv, bv2, wal8)


_IOTA = None


def _shifted(buf_ref, iota, shift):
  return plsc.load_gather(buf_ref, [iota + (16 - shift)])


def _seg_combine(ids, vals, ibuf, vbuf, iota, is_max):
  neutral = jnp.float32(-jnp.inf) if is_max else jnp.float32(0.0)
  v = vals
  for s in (1, 2, 4, 8):
    vbuf[pl.ds(16, 16)] = v
    sv = _shifted(vbuf, iota, s)
    si = _shifted(ibuf, iota, s)
    contrib = jnp.where(si == ids, sv, neutral)
    v = jnp.maximum(v, contrib) if is_max else v + contrib
  return v


def _k2_body(bid_hbm, at_hbm, alpha_hbm, cnt_hbm, boff_hbm,
             bid_v, a_v, ibuf, vbuf, mloc, sloc, cloc, tmp, tmpc,
             red, credv, mful_v, sful_v, alpha_st, cnt_v, boff_v,
             mbuf_sh, cbuf_sh, mful_sh, sful_sh, cful_sh):
  c = lax.axis_index("c")
  s = lax.axis_index("s")
  iota = lax.iota(jnp.int32, 16)
  base = s * CHUNK

  pltpu.sync_copy(bid_hbm.at[pl.ds(base, CHUNK)], bid_v.at[pl.ds(0, CHUNK)])
  bid_v[pl.ds(CHUNK, 16)] = jnp.full((16,), _SENT, jnp.int32)
  for d in range(D):
    pltpu.sync_copy(at_hbm.at[pl.ds(d * NP + base, CHUNK)],
                    a_v.at[pl.ds(d * CHUNK, CHUNK)])

  ibuf[pl.ds(0, 16)] = jnp.full((16,), -1, jnp.int32)
  vbuf[pl.ds(0, 16)] = jnp.zeros((16,), jnp.float32)

  def _init(i, _):
    off = i * 16
    for d in range(D):
      mloc[d, pl.ds(off, 16)] = jnp.full((16,), -jnp.inf, jnp.float32)
      sloc[d, pl.ds(off, 16)] = jnp.zeros((16,), jnp.float32)
    cloc[pl.ds(off, 16)] = jnp.zeros((16,), jnp.float32)
    return 0
  lax.fori_loop(0, BP // 16, _init, 0)

  def _s1(v, _):
    b0 = v * 16
    ids = bid_v[pl.ds(b0, 16)]
    ids_nx = bid_v[pl.ds(b0 + 1, 16)]
    lastm = jnp.logical_or(ids != ids_nx, iota == 15)
    ibuf[pl.ds(16, 16)] = ids
    cv = _seg_combine(ids, jnp.ones((16,), jnp.float32), ibuf, vbuf, iota,
                      is_max=False)
    plsc.addupdate_scatter(cloc, [ids], cv, mask=lastm)
    for d in range(D):
      dsp = jnp.full((16,), d, jnp.int32)
      av = a_v[pl.ds(d * CHUNK + b0, 16)]
      mv = _seg_combine(ids, av, ibuf, vbuf, iota, is_max=True)
      cur = plsc.load_gather(mloc, [dsp, ids])
      plsc.store_scatter(mloc, [dsp, ids], jnp.maximum(cur, mv), mask=lastm)
    return 0
  lax.fori_loop(0, NV, _s1, 0)

  pltpu.sync_copy(mloc, mbuf_sh.at[s])
  pltpu.sync_copy(cloc, cbuf_sh.at[s])
  plsc.subcore_barrier()

  def _initred(i, _):
    off = i * 16
    for d in range(D):
      red[d, pl.ds(off, 16)] = jnp.full((16,), -jnp.inf, jnp.float32)
    credv[pl.ds(off, 16)] = jnp.zeros((16,), jnp.float32)
    return 0
  lax.fori_loop(0, 8, _initred, 0)
  win = s * 128
  for u in range(NT):
    pltpu.sync_copy(mbuf_sh.at[u], tmp)
    pltpu.sync_copy(cbuf_sh.at[u], tmpc)
    for d in range(D):
      for vv in range(8):
        o = vv * 16
        red[d, pl.ds(o, 16)] = jnp.maximum(
            red[d, pl.ds(o, 16)], tmp[d, pl.ds(win + o, 16)])
    for vv in range(8):
      o = vv * 16
      credv[pl.ds(o, 16)] = credv[pl.ds(o, 16)] + tmpc[pl.ds(win + o, 16)]
  pltpu.sync_copy(red, mful_sh.at[s])
  pltpu.sync_copy(credv, cful_sh.at[s])
  plsc.subcore_barrier()
  pltpu.sync_copy(mful_sh, mful_v)

  @pl.when(s == 0)
  def _():
    pltpu.sync_copy(cful_sh, cnt_hbm.at[c])

  @pl.when(jnp.logical_and(c == 0, s == 0))
  def _():
    pltpu.sync_copy(cful_sh, cnt_v)

    def _pfx(i, carry):
      q = lax.shift_right_logical(i, 3)
      o = jnp.bitwise_and(i, 7) * 16
      v = cnt_v[q, pl.ds(o, 16)]
      ex = plsc.cumsum(v) - v + carry
      boff_v[pl.ds(i * 16, 16)] = ex.astype(jnp.int32)
      return carry + jnp.sum(v)
    lax.fori_loop(0, BP // 16, _pfx, jnp.float32(0.0))
    pltpu.sync_copy(boff_v, boff_hbm)

  def _s2(v, _):
    b0 = v * 16
    ids = bid_v[pl.ds(b0, 16)]
    ids_nx = bid_v[pl.ds(b0 + 1, 16)]
    lastm = jnp.logical_or(ids != ids_nx, iota == 15)
    ibuf[pl.ds(16, 16)] = ids
    q = lax.shift_right_logical(ids, 7)
    r7 = jnp.bitwise_and(ids, 127)
    for d in range(D):
      dsp = jnp.full((16,), d, jnp.int32)
      m = plsc.load_gather(mful_v, [q, dsp, r7])
      ev = jnp.exp(a_v[pl.ds(d * CHUNK + b0, 16)] - m)
      ev = _seg_combine(ids, ev, ibuf, vbuf, iota, is_max=False)
      plsc.addupdate_scatter(sloc, [dsp, ids], ev, mask=lastm)
    return 0
  lax.fori_loop(0, NV, _s2, 0)

  pltpu.sync_copy(sloc, mbuf_sh.at[s])
  plsc.subcore_barrier()

  def _initred2(i, _):
    off = i * 16
    for d in range(D):
      red[d, pl.ds(off, 16)] = jnp.zeros((16,), jnp.float32)
    return 0
  lax.fori_loop(0, 8, _initred2, 0)
  for u in range(NT):
    pltpu.sync_copy(mbuf_sh.at[u], tmp)
    for d in range(D):
      for vv in range(8):
        o = vv * 16
        red[d, pl.ds(o, 16)] = red[d, pl.ds(o, 16)] + tmp[d, pl.ds(win + o, 16)]
  pltpu.sync_copy(red, sful_sh.at[s])
  plsc.subcore_barrier()
  pltpu.sync_copy(sful_sh, sful_v)

  coff = c * HALF

  def _s3(v, _):
    b0 = coff + v * 16
    ids = bid_v[pl.ds(b0, 16)]
    q = lax.shift_right_logical(ids, 7)
    r7 = jnp.bitwise_and(ids, 127)
    for d in range(D):
      dsp = jnp.full((16,), d, jnp.int32)
      m = plsc.load_gather(mful_v, [q, dsp, r7])
      sv = plsc.load_gather(sful_v, [q, dsp, r7])
      al = jnp.exp(a_v[pl.ds(d * CHUNK + b0, 16)] - m) / (sv + 1e-9)
      alpha_st[pl.ds(d * HALF + v * 16, 16)] = al
    return 0
  lax.fori_loop(0, NV // 2, _s3, 0)
  for d in range(D):
    pltpu.sync_copy(alpha_st.at[pl.ds(d * HALF, HALF)],
                    alpha_hbm.at[pl.ds(d * NP + base + coff, HALF)])


def _k2(bid_p, a_t_flat):
  mesh = plsc.VectorSubcoreMesh(core_axis_name="c", subcore_axis_name="s")
  f = pl.kernel(
      _k2_body,
      out_type=(
          jax.ShapeDtypeStruct((8 * NP,), jnp.float32),
          jax.ShapeDtypeStruct((2, NT, 128), jnp.float32),
          jax.ShapeDtypeStruct((BP,), jnp.int32),
      ),
      mesh=mesh,
      scratch_types=[
          pltpu.VMEM((CHUNK + 16,), jnp.int32),
          pltpu.VMEM((D * CHUNK,), jnp.float32),
          pltpu.VMEM((32,), jnp.int32),
          pltpu.VMEM((32,), jnp.float32),
          pltpu.VMEM((D, BP), jnp.float32),
          pltpu.VMEM((D, BP), jnp.float32),
          pltpu.VMEM((BP,), jnp.float32),
          pltpu.VMEM((D, BP), jnp.float32),
          pltpu.VMEM((BP,), jnp.float32),
          pltpu.VMEM((D, 128), jnp.float32),
          pltpu.VMEM((128,), jnp.float32),
          pltpu.VMEM((NT, D, 128), jnp.float32),
          pltpu.VMEM((NT, D, 128), jnp.float32),
          pltpu.VMEM((D * HALF,), jnp.float32),
          pltpu.VMEM((NT, 128), jnp.float32),
          pltpu.VMEM((BP,), jnp.int32),
          pltpu.VMEM_SHARED((NT, D, BP), jnp.float32),
          pltpu.VMEM_SHARED((NT, BP), jnp.float32),
          pltpu.VMEM_SHARED((NT, D, 128), jnp.float32),
          pltpu.VMEM_SHARED((NT, D, 128), jnp.float32),
          pltpu.VMEM_SHARED((NT, 128), jnp.float32),
      ],
      compiler_params=pltpu.CompilerParams(needs_layout_passes=False),
  )
  return f(bid_p, a_t_flat)


def _k3_body(hv_ref, al_ref, wa_ref, ba_ref, out_ref):
  h = hv_ref[...].astype(jnp.bfloat16)
  al = jnp.transpose(al_ref[...], (1, 0))
  wa = wa_ref[...].astype(jnp.bfloat16)
  ba = ba_ref[...]
  for d in range(D):
    p = jnp.dot(h, wa[d], preferred_element_type=jnp.float32) + ba[d]
    p = jnp.maximum(p, 0.0) * al[:, d:d + 1]
    out_ref[:, d * F:(d + 1) * F] = p


def _k3(hv, alpha_t, wa, ba):
  return pl.pallas_call(
      _k3_body,
      grid=(NBLK,),
      in_specs=[
          pl.BlockSpec((BLK, F), lambda i: (i, 0)),
          pl.BlockSpec((8, BLK), lambda i: (0, i)),
          pl.BlockSpec((D, F, F), lambda i: (0, 0, 0)),
          pl.BlockSpec((D, F), lambda i: (0, 0)),
      ],
      out_specs=pl.BlockSpec((BLK, D * F), lambda i: (i, 0)),
      out_shape=jax.ShapeDtypeStruct((NP, D * F), jnp.float32),
      compiler_params=pltpu.CompilerParams(
          dimension_semantics=("arbitrary",)),
  )(hv, alpha_t, wa, ba)


GPC = B // 32
ACCR = GPC + 1
AW = F + D * F


def _k4_body(bid_hbm, boff_hbm, hv_hbm, watt_hbm, pool_hbm,
             bo_v, bidv_r, locb, bh0, bw0, bh1, bw1, acc,
             semb0, semh0, semw0, semb1, semh1, semw1):
  c = lax.axis_index("c")
  s = lax.axis_index("s")
  iota = lax.iota(jnp.int32, 16)
  wid = c * NT + s
  g0 = pl.multiple_of(wid * GPC, GPC)

  pltpu.sync_copy(boff_hbm.at[pl.ds(g0, 40)], bo_v)
  start = jnp.min(plsc.load_gather(bo_v, [jnp.zeros((16,), jnp.int32)]))
  end = jnp.min(plsc.load_gather(bo_v, [jnp.full((16,), GPC, jnp.int32)]))
  start = pl.multiple_of(jnp.bitwise_and(start, ~7), 8)
  end = jnp.bitwise_and(end + 7, ~7)
  nch = (end - start + 23) // 24
  nch2 = lax.shift_right_logical(nch + 1, 1)

  def _zr(r, _):
    for k in range(AW // 16):
      acc[r, pl.ds(k * 16, 16)] = jnp.zeros((16,), jnp.float32)
    return 0
  lax.fori_loop(0, ACCR, _zr, 0)

  bufs = ((bidv_r.at[pl.ds(0, 24)], bh0, bw0, semb0, semh0, semw0),
          (bidv_r.at[pl.ds(24, 24)], bh1, bw1, semb1, semh1, semw1))

  def _issue(i, b):
    bb, hb, wb, sb, sh, sw = bufs[b]
    r0 = pl.multiple_of(start + i * 24, 8)
    pltpu.async_copy(bid_hbm.at[pl.ds(r0, 24)], bb, sb)
    pltpu.async_copy(hv_hbm.at[pl.ds(r0, 24)], hb, sh)
    pltpu.async_copy(watt_hbm.at[pl.ds(r0, 24)], wb, sw)

  def _wait(b):
    bb, hb, wb, sb, sh, sw = bufs[b]
    pltpu.make_async_copy(bid_hbm.at[pl.ds(0, 24)], bb, sb).wait()
    pltpu.make_async_copy(hv_hbm.at[pl.ds(0, 24)], hb, sh).wait()
    pltpu.make_async_copy(watt_hbm.at[pl.ds(0, 24)], wb, sw).wait()

  def _compute(b):
    bb, hb, wb, _, _, _ = bufs[b]
    for hh in (0, 1):
      loc = bb[pl.ds(8 * hh, 16)] - g0
      loc = jnp.where(
          jnp.logical_or(loc < 0, loc >= GPC), jnp.int32(GPC), loc)
      locb[pl.ds(8 * hh, 16)] = loc

    def _row(j, _):
      rowv = plsc.load_gather(locb, [jnp.zeros((16,), jnp.int32) + j])
      for k in range(F // 16):
        plsc.addupdate_scatter(
            acc, [rowv, iota + k * 16], hb[j, pl.ds(k * 16, 16)])
      for k in range(D * F // 16):
        plsc.addupdate_scatter(
            acc, [rowv, iota + (F + k * 16)], wb[j, pl.ds(k * 16, 16)])
      return 0
    lax.fori_loop(0, 24, _row, 0)

  _issue(0, 0)

  def _pair(g, _):
    i = g * 2
    _wait(0)
    _issue(i + 1, 1)
    _compute(0)
    _wait(1)
    _issue(i + 2, 0)
    _compute(1)
    return 0
  lax.fori_loop(0, nch2, _pair, 0)
  _wait(0)

  pltpu.sync_copy(acc.at[pl.ds(0, GPC)], pool_hbm.at[pl.ds(g0, GPC)])


def _k4(bid_p, boffs, hv, watt):
  mesh = plsc.VectorSubcoreMesh(core_axis_name="c", subcore_axis_name="s")
  f = pl.kernel(
      _k4_body,
      out_type=jax.ShapeDtypeStruct((B, AW), jnp.float32),
      mesh=mesh,
      scratch_types=[
          pltpu.VMEM((40,), jnp.int32),
          pltpu.VMEM((48,), jnp.int32),
          pltpu.VMEM((32,), jnp.int32),
          pltpu.VMEM((24, F), jnp.float32),
          pltpu.VMEM((24, D * F), jnp.float32),
          pltpu.VMEM((24, F), jnp.float32),
          pltpu.VMEM((24, D * F), jnp.float32),
          pltpu.VMEM((ACCR, AW), jnp.float32),
          pltpu.SemaphoreType.DMA,
          pltpu.SemaphoreType.DMA,
          pltpu.SemaphoreType.DMA,
          pltpu.SemaphoreType.DMA,
          pltpu.SemaphoreType.DMA,
          pltpu.SemaphoreType.DMA,
      ],
      compiler_params=pltpu.CompilerParams(needs_layout_passes=False),
  )
  return f(bid_p, boffs, hv, watt)


def _k5_body(pool_ref, cnt_ref, gk_ref, grk_ref, gb_ref, out_ref):
  pool = pool_ref[...]
  hvsum = pool[:, :F]
  cnt = jnp.transpose(cnt_ref[...], (1, 0))
  gk = gk_ref[...]
  grk = grk_ref[...]
  gb = gb_ref[...]
  hm = hvsum / jnp.maximum(cnt, 1.0)
  for i in range(D):
    mm = pool[:, F + i * F:F + (i + 1) * F]
    mm = jnp.where(mm > 0, mm, jnp.exp(jnp.minimum(mm, 0.0)) - 1.0)
    mx = jnp.dot(mm, gk, preferred_element_type=jnp.float32) + gb[0]
    mh = jnp.dot(hm, grk, preferred_element_type=jnp.float32) + gb[1]
    z = jax.nn.sigmoid(mx[:, :F] + mh[:, :F])
    r = jax.nn.sigmoid(mx[:, F:2 * F] + mh[:, F:2 * F])
    hh = jnp.tanh(mx[:, 2 * F:] + r * mh[:, 2 * F:])
    hm = jnp.maximum(z * hm + (1.0 - z) * hh, 0.0)
  out_ref[...] = hm


def _k5(pool, cnt_row, gru_k, gru_rk, gru_b):
  return pl.pallas_call(
      _k5_body,
      in_specs=[
          pl.BlockSpec((B, AW), lambda: (0, 0)),
          pl.BlockSpec((1, B), lambda: (0, 0)),
          pl.BlockSpec((F, 3 * F), lambda: (0, 0)),
          pl.BlockSpec((F, 3 * F), lambda: (0, 0)),
          pl.BlockSpec((2, 3 * F), lambda: (0, 0)),
      ],
      out_specs=pl.BlockSpec((B, F), lambda: (0, 0)),
      out_shape=jax.ShapeDtypeStruct((B, F), jnp.float32),
  )(pool, cnt_row, gru_k, gru_rk, gru_b)


def kernel(count_nodes, nodes, batch_id, Wv, bv, Wa, ba, Wal, bal,
           gru_k, gru_rk, gru_b):
  del count_nodes, bal
  nodes_p = jnp.concatenate(
      [nodes, jnp.zeros((NP - N, F), jnp.float32)], axis=0)
  bid_p = jnp.concatenate(
      [batch_id.astype(jnp.int32), jnp.full((NP - N,), B, jnp.int32)])
  wal8 = jnp.zeros((8, F), jnp.float32).at[:D].set(Wal[:, F:, 0])
  bv2 = bv.reshape(1, F)

  hv, a_t = _k1(nodes_p, Wv, bv2, wal8)
  alpha_flat, cnt2, boffs = _k2(bid_p, a_t.reshape(-1))
  watt = _k3(hv, alpha_flat.reshape(8, NP), Wa, ba)
  pool = _k4(bid_p, boffs, hv, watt)
  cnt_row = cnt2.reshape(2, NT * 128)[0:1, :B]
  return _k5(pool, cnt_row, gru_k, gru_rk, gru_b)
